# baseline (speedup 1.0000x reference)
import functools
import math

import jax
import jax.numpy as jnp
from jax import lax
from jax.experimental import pallas as pl
from jax.experimental.pallas import tpu as pltpu
from jax.experimental.pallas import tpu_sc as plsc

B, L, D = 4096, 50, 64
N = B * L
NC, NS = 2, 16
NW = NC * NS
CHUNK = 2 * L
RPW = N // NW
CPW = RPW // CHUNK
G = D // 16


def _pe_body(o_ref):
    row = lax.broadcasted_iota(jnp.int32, (CHUNK, D), 0)
    col = lax.broadcasted_iota(jnp.int32, (CHUNK, D), 1).astype(jnp.float32)
    pos = (row % L).astype(jnp.float32)
    inv_freq = jnp.exp(col * (-math.log(10000.0) / D))
    o_ref[...] = jnp.sin(pos * inv_freq)


def _make_pe():
    return pl.pallas_call(
        _pe_body,
        out_shape=jax.ShapeDtypeStruct((CHUNK, D), jnp.float32),
    )()


NBUF = 8
LEAD = 4
NGRP = CPW // NBUF


def _sc_body(table_hbm, idx_hbm, pe_hbm, out_hbm, idx_v, pe_v, buf, sem_g, sem_s):
    w = lax.axis_index("s") * NC + lax.axis_index("c")
    pltpu.sync_copy(idx_hbm.at[pl.ds(w * CPW, CPW)], idx_v)
    pltpu.sync_copy(pe_hbm, pe_v)

    for b in range(LEAD):
        pltpu.async_copy(table_hbm.at[idx_v.at[b]], buf.at[b], sem_g.at[b])

    def group(g, carry):
        for b in range(NBUF):
            j = g * NBUF + b
            pltpu.make_async_copy(
                table_hbm.at[idx_v.at[j]], buf.at[b], sem_g.at[b]).wait()

            def addrow(r, c2):
                for k in range(G):
                    sl = pl.ds(k * 16, 16)
                    buf[b, r, sl] = buf[b, r, sl] + pe_v[r, sl]
                return c2

            lax.fori_loop(0, CHUNK, addrow, 0)
            pltpu.async_copy(
                buf.at[b, :, pl.ds(0, D)], out_hbm.at[w * CPW + j], sem_s.at[b])
            jn = j + LEAD
            jb = (b + LEAD) % NBUF

            @pl.when(jn < CPW)
            def _():
                @pl.when(jn >= NBUF)
                def _():
                    pltpu.make_async_copy(
                        buf.at[jb, :, pl.ds(0, D)], out_hbm.at[w * CPW + j],
                        sem_s.at[jb]).wait()

                pltpu.async_copy(
                    table_hbm.at[idx_v.at[jn]], buf.at[jb], sem_g.at[jb])
        return carry

    lax.fori_loop(0, NGRP, group, 0)
    for b in range(NBUF):
        pltpu.make_async_copy(
            buf.at[b, :, pl.ds(0, D)], out_hbm.at[b], sem_s.at[b]).wait()


_sc_gather = functools.partial(
    pl.kernel,
    mesh=plsc.VectorSubcoreMesh(core_axis_name="c", subcore_axis_name="s"),
    out_type=jax.ShapeDtypeStruct((N // CHUNK, CHUNK, D), jnp.float32),
    scratch_types=[
        pltpu.VMEM((CPW, CHUNK), jnp.int32),
        pltpu.VMEM((CHUNK, D), jnp.float32),
        pltpu.VMEM((NBUF, CHUNK, 2 * D), jnp.float32),
        pltpu.SemaphoreType.DMA((NBUF,)),
        pltpu.SemaphoreType.DMA((NBUF,)),
    ],
    compiler_params=pltpu.CompilerParams(use_tc_tiling_on_sc=False),
)(_sc_body)


def kernel(input_ids, table):
    idx = input_ids.reshape(N // CHUNK, CHUNK).astype(jnp.int32)
    tpad = jnp.pad(table, ((0, 0), (0, 2 * D - table.shape[1])))
    pe = _make_pe()
    out = _sc_gather(tpad, idx, pe)
    return out.reshape(B, L, D)


# speedup vs baseline: 1.0143x; 1.0143x over previous
import functools
import math

import jax
import jax.numpy as jnp
from jax import lax
from jax.experimental import pallas as pl
from jax.experimental.pallas import tpu as pltpu
from jax.experimental.pallas import tpu_sc as plsc

B, L, D = 4096, 50, 64
N = B * L
NC, NS = 2, 16
NW = NC * NS
CHUNK = 2 * L
RPW = N // NW
CPW = RPW // CHUNK
G = D // 16


def _pe_body(o_ref):
    row = lax.broadcasted_iota(jnp.int32, (CHUNK, D), 0)
    col = lax.broadcasted_iota(jnp.int32, (CHUNK, D), 1).astype(jnp.float32)
    pos = (row % L).astype(jnp.float32)
    inv_freq = jnp.exp(col * (-math.log(10000.0) / D))
    o_ref[...] = jnp.sin(pos * inv_freq)


def _make_pe():
    return pl.pallas_call(
        _pe_body,
        out_shape=jax.ShapeDtypeStruct((CHUNK, D), jnp.float32),
    )()


TBLK = 2048
_TGRID = (VOCAB := 1000000) // TBLK + 1


def _tr_body(tt_ref, o_ref):
    t = tt_ref[...].T
    o_ref[...] = jnp.concatenate([t, t], axis=1)


def _relayout_table(tt):
    return pl.pallas_call(
        _tr_body,
        grid=(_TGRID,),
        in_specs=[pl.BlockSpec((D, TBLK), lambda i: (0, i))],
        out_specs=pl.BlockSpec((TBLK, 2 * D), lambda i: (i, 0)),
        out_shape=jax.ShapeDtypeStruct((VOCAB, 2 * D), jnp.float32),
    )(tt)


NBUF = 8
LEAD = 4
NGRP = CPW // NBUF


def _sc_body(table_hbm, idx_hbm, pe_hbm, out_hbm, idx_v, pe_v, buf, sem_g, sem_s):
    w = lax.axis_index("s") * NC + lax.axis_index("c")
    pltpu.sync_copy(idx_hbm.at[pl.ds(w * CPW, CPW)], idx_v)
    pltpu.sync_copy(pe_hbm, pe_v)

    for b in range(LEAD):
        pltpu.async_copy(table_hbm.at[idx_v.at[b]], buf.at[b], sem_g.at[b])

    def group(g, carry):
        for b in range(NBUF):
            j = g * NBUF + b
            pltpu.make_async_copy(
                table_hbm.at[idx_v.at[j]], buf.at[b], sem_g.at[b]).wait()

            def addrow(r, c2):
                for k in range(G):
                    sl = pl.ds(k * 16, 16)
                    buf[b, r, sl] = buf[b, r, sl] + pe_v[r, sl]
                return c2

            lax.fori_loop(0, CHUNK, addrow, 0)
            pltpu.async_copy(
                buf.at[b, :, pl.ds(0, D)], out_hbm.at[w * CPW + j], sem_s.at[b])
            jn = j + LEAD
            jb = (b + LEAD) % NBUF

            @pl.when(jn < CPW)
            def _():
                @pl.when(jn >= NBUF)
                def _():
                    pltpu.make_async_copy(
                        buf.at[jb, :, pl.ds(0, D)], out_hbm.at[w * CPW + j],
                        sem_s.at[jb]).wait()

                pltpu.async_copy(
                    table_hbm.at[idx_v.at[jn]], buf.at[jb], sem_g.at[jb])
        return carry

    lax.fori_loop(0, NGRP, group, 0)
    for b in range(NBUF):
        pltpu.make_async_copy(
            buf.at[b, :, pl.ds(0, D)], out_hbm.at[b], sem_s.at[b]).wait()


_sc_gather = functools.partial(
    pl.kernel,
    mesh=plsc.VectorSubcoreMesh(core_axis_name="c", subcore_axis_name="s"),
    out_type=jax.ShapeDtypeStruct((N // CHUNK, CHUNK, D), jnp.float32),
    scratch_types=[
        pltpu.VMEM((CPW, CHUNK), jnp.int32),
        pltpu.VMEM((CHUNK, D), jnp.float32),
        pltpu.VMEM((NBUF, CHUNK, 2 * D), jnp.float32),
        pltpu.SemaphoreType.DMA((NBUF,)),
        pltpu.SemaphoreType.DMA((NBUF,)),
    ],
    compiler_params=pltpu.CompilerParams(use_tc_tiling_on_sc=False),
)(_sc_body)


def kernel(input_ids, table):
    idx = input_ids.reshape(N // CHUNK, CHUNK).astype(jnp.int32)
    tbl = _relayout_table(table.T)
    pe = _make_pe()
    out = _sc_gather(tbl, idx, pe)
    return out.reshape(B, L, D)


# speedup vs baseline: 1.0867x; 1.0714x over previous
import functools
import math

import jax
import jax.numpy as jnp
from jax import lax
from jax.experimental import pallas as pl
from jax.experimental.pallas import tpu as pltpu
from jax.experimental.pallas import tpu_sc as plsc

B, L, D = 4096, 50, 64
N = B * L
NC, NS = 2, 16
NW = NC * NS
CHUNK = 2 * L
RPW = N // NW
CPW = RPW // CHUNK
G = D // 16


def _pe_body(o_ref):
    row = lax.broadcasted_iota(jnp.int32, (CHUNK, D), 0)
    col = lax.broadcasted_iota(jnp.int32, (CHUNK, D), 1).astype(jnp.float32)
    pos = (row % L).astype(jnp.float32)
    inv_freq = jnp.exp(col * (-math.log(10000.0) / D))
    o_ref[...] = jnp.sin(pos * inv_freq)


def _make_pe():
    return pl.pallas_call(
        _pe_body,
        out_shape=jax.ShapeDtypeStruct((CHUNK, D), jnp.float32),
    )()


TBLK = 2048
_TGRID = (VOCAB := 1000000) // TBLK + 1


def _tr_body(tt_ref, o_ref):
    blk = tt_ref[...]
    eye = (lax.broadcasted_iota(jnp.int32, (D, D), 0)
           == lax.broadcasted_iota(jnp.int32, (D, D), 1)).astype(jnp.float32)
    t = lax.dot_general(blk, eye, (((0,), (0,)), ((), ())),
                        preferred_element_type=jnp.float32)
    o_ref[:, 0:D] = t


def _relayout_table(tt):
    return pl.pallas_call(
        _tr_body,
        grid=(_TGRID,),
        in_specs=[pl.BlockSpec((D, TBLK), lambda i: (0, i))],
        out_specs=pl.BlockSpec((TBLK, 2 * D), lambda i: (i, 0)),
        out_shape=jax.ShapeDtypeStruct((VOCAB, 2 * D), jnp.float32),
    )(tt)


NBUF = 4
LEAD = 2
NGRP = CPW // NBUF
HCH = CHUNK // 2


def _sc_body(table_hbm, idx_hbm, pe_hbm, out_hbm,
             idx_v, pe_v, buf, obuf, sem_g, sem_s):
    w = lax.axis_index("s") * NC + lax.axis_index("c")
    pltpu.sync_copy(idx_hbm.at[pl.ds(w * CPW, CPW)], idx_v)
    pltpu.sync_copy(pe_hbm, pe_v)

    for b in range(LEAD):
        pltpu.async_copy(table_hbm.at[idx_v.at[b]], buf.at[b], sem_g.at[b])

    def group(g, carry):
        for b in range(NBUF):
            j = g * NBUF + b
            pltpu.make_async_copy(
                table_hbm.at[idx_v.at[j]], buf.at[b], sem_g.at[b]).wait()

            def addrow(r, c2):
                r2 = r // 2
                p = (r % 2) * D
                for k in range(G):
                    src = pl.ds(k * 16, 16)
                    dst = pl.ds(p + k * 16, 16)
                    obuf[b, r2, dst] = buf[b, r, src] + pe_v[r2, dst]
                return c2

            lax.fori_loop(0, CHUNK, addrow, 0)
            row0 = (w * CPW + j) * HCH
            pltpu.async_copy(obuf.at[b], out_hbm.at[pl.ds(row0, HCH)],
                             sem_s.at[b])
            jn = j + LEAD
            jb = (b + LEAD) % NBUF

            @pl.when(jn < CPW)
            def _():
                @pl.when(jn >= NBUF)
                def _():
                    pltpu.make_async_copy(
                        obuf.at[jb], out_hbm.at[pl.ds(row0, HCH)],
                        sem_s.at[jb]).wait()

                pltpu.async_copy(
                    table_hbm.at[idx_v.at[jn]], buf.at[jb], sem_g.at[jb])
        return carry

    lax.fori_loop(0, NGRP, group, 0)
    for b in range(NBUF):
        pltpu.make_async_copy(
            obuf.at[b], out_hbm.at[pl.ds(0, HCH)], sem_s.at[b]).wait()


_sc_gather = functools.partial(
    pl.kernel,
    mesh=plsc.VectorSubcoreMesh(core_axis_name="c", subcore_axis_name="s"),
    out_type=jax.ShapeDtypeStruct((N // 2, 2 * D), jnp.float32),
    scratch_types=[
        pltpu.VMEM((CPW, CHUNK), jnp.int32),
        pltpu.VMEM((L, 2 * D), jnp.float32),
        pltpu.VMEM((NBUF, CHUNK, 2 * D), jnp.float32),
        pltpu.VMEM((NBUF, HCH, 2 * D), jnp.float32),
        pltpu.SemaphoreType.DMA((NBUF,)),
        pltpu.SemaphoreType.DMA((NBUF,)),
    ],
    compiler_params=pltpu.CompilerParams(use_tc_tiling_on_sc=False),
)(_sc_body)


def kernel(input_ids, table):
    idx = input_ids.reshape(N // CHUNK, CHUNK).astype(jnp.int32)
    tbl = _relayout_table(table.T)
    pe = _make_pe().reshape(L, 2 * D)
    out = _sc_gather(tbl, idx, pe)
    return out.reshape(B, L, D)
